# Initial kernel scaffold; baseline (speedup 1.0000x reference)
import jax
import jax.numpy as jnp
from jax.experimental import pallas as pl


def kernel(x, table):
    raise NotImplementedError("write your pallas kernel here")


# speedup vs baseline: 1.9672x; 1.9672x over previous
import jax
import jax.numpy as jnp
from jax import lax
from jax.experimental import pallas as pl
from jax.experimental.pallas import tpu as pltpu
from jax.experimental.pallas import tpu_sc as plsc

BATCH = 16384
N_FIELDS = 26
EMBED_DIM = 32

NC, NS = 2, 16
NW = NC * NS
ROWS_PER_W = BATCH // NW
CHUNK = 64
N_CHUNKS = ROWS_PER_W // CHUNK
IDX_PER_CHUNK = CHUNK * N_FIELDS
IDX_W = 128
IDX_ROWS = IDX_PER_CHUNK // IDX_W
HALVES = EMBED_DIM // 16


_GATHER_DNUMS = lax.GatherDimensionNumbers(
    offset_dims=(), collapsed_slice_dims=(0,), start_index_map=(0,))


def _shuffle16(v, idx):
    return lax.gather(v, idx[:, None], _GATHER_DNUMS, (1,),
                      mode=lax.GatherScatterMode.PROMISE_IN_BOUNDS)


def _lane_sum(v):
    lanes = lax.iota(jnp.int32, 16)
    for s in (8, 4, 2, 1):
        v = v + _shuffle16(v, lanes ^ s)
    return v


def _fast_rsqrt(v):
    i = lax.bitcast_convert_type(v, jnp.int32)
    i = jnp.int32(0x5F3759DF) - (i >> 1)
    y = lax.bitcast_convert_type(i, jnp.float32)
    for _ in range(3):
        y = y * (1.5 - 0.5 * v * y * y)
    return y


def _sc_body(x_hbm, table_hbm, out_hbm, idx_v, rows_v, sem):
    wid = lax.axis_index("s") * NC + lax.axis_index("c")

    pltpu.sync_copy(x_hbm.at[wid], idx_v)

    def chunk_body(c, carry):
        copies = [
            pltpu.async_copy(
                table_hbm.at[idx_v.at[c * IDX_ROWS + j]],
                rows_v.at[pl.ds(j * IDX_W, IDX_W)],
                sem,
            )
            for j in range(IDX_ROWS)
        ]
        for cp in copies:
            cp.wait()

        def row_body(i, carry2):
            base = i * N_FIELDS
            acc = jnp.zeros((16,), jnp.float32)
            for r in range(N_FIELDS):
                for h in range(HALVES):
                    v = rows_v[base + r, pl.ds(h * 16, 16)]
                    acc = acc + v * v
            ssq = jnp.maximum(_lane_sum(acc), 1e-24)
            scale = _fast_rsqrt(ssq)
            for r in range(N_FIELDS):
                for h in range(HALVES):
                    sl = (base + r, pl.ds(h * 16, 16))
                    rows_v[sl] = rows_v[sl] * scale
            return carry2

        lax.fori_loop(0, CHUNK, row_body, 0)

        out_row0 = (wid * ROWS_PER_W + c * CHUNK) * N_FIELDS
        pltpu.sync_copy(rows_v, out_hbm.at[pl.ds(out_row0, IDX_PER_CHUNK)])
        return carry

    lax.fori_loop(0, N_CHUNKS, chunk_body, 0)


def kernel(x, table):
    x2 = x.reshape(NW, ROWS_PER_W * N_FIELDS // IDX_W, IDX_W)
    out = pl.kernel(
        _sc_body,
        out_type=jax.ShapeDtypeStruct((BATCH * N_FIELDS, EMBED_DIM),
                                      jnp.float32),
        mesh=plsc.VectorSubcoreMesh(core_axis_name="c", subcore_axis_name="s"),
        compiler_params=pltpu.CompilerParams(use_tc_tiling_on_sc=False),
        scratch_types=[
            pltpu.VMEM((ROWS_PER_W * N_FIELDS // IDX_W, IDX_W), jnp.int32),
            pltpu.VMEM((IDX_PER_CHUNK, EMBED_DIM), jnp.float32),
            pltpu.SemaphoreType.DMA,
        ],
    )(x2, table)
    return out.reshape(BATCH, N_FIELDS * EMBED_DIM)
